# Initial kernel scaffold; baseline (speedup 1.0000x reference)
import jax
import jax.numpy as jnp
from jax.experimental import pallas as pl


def kernel(edge_index, edge_type, edge_time, query_triple, query_table, rel0, rel1, tw, tb, W0, b0, W1, b1, M0, mb0, M1, mb1):
    raise NotImplementedError("write your pallas kernel here")


# speedup vs baseline: 23.1994x; 23.1994x over previous
import functools

import jax
import jax.numpy as jnp
from jax import lax
from jax.experimental import pallas as pl
from jax.experimental.pallas import tpu as pltpu
from jax.experimental.pallas import tpu_sc as plsc

N = 10000
E = 160000
D = 64
R = 100
B = 4
NEG = 33
HIST = 10

NC = 2
NS = 16
CHUNK = 80
EPT = E // NS
NCHUNKS = EPT // CHUNK
ROWS_PT = 1248
ROWS_REM = 2 * N - NS * ROWS_PT
ZROWS = 312

NEGP = 40


def _prep_body(rel0_ref, rel1_ref, tw_ref, tb_ref, qt_ref, rn_ref,
               comb_ref, q_ref):
    t_ids = lax.broadcasted_iota(jnp.int32, (HIST, D), 0).astype(jnp.float32)
    te = jnp.cos(t_ids * tw_ref[0, :][None, :] + tb_ref[0, :][None, :])
    comb_ref[0] = rel0_ref[...][:, None, :] + te[None, :, :]
    comb_ref[1] = rel1_ref[...][:, None, :] + te[None, :, :]
    rids = lax.broadcasted_iota(jnp.int32, (B, R), 1)
    onehot = (rids == rn_ref[...]).astype(jnp.float32)
    q_ref[...] = jnp.dot(onehot, qt_ref[...],
                         preferred_element_type=jnp.float32)


def _prep(rel0, rel1, tw, tb, query_table, rn0):
    comb4, query = pl.pallas_call(
        _prep_body,
        out_shape=(
            jax.ShapeDtypeStruct((2, R, HIST, D), jnp.float32),
            jax.ShapeDtypeStruct((B, D), jnp.float32),
        ),
    )(rel0, rel1, tw.reshape(1, D), tb.reshape(1, D), query_table,
      rn0.reshape(B, 1))
    return comb4.reshape(2 * R * HIST, D), query


BN = 2000


def _boundary_body(h_ref, q_ref, out_ref):
    b = pl.program_id(0)
    j = pl.program_id(1)
    hb = h_ref[b]
    rows = lax.broadcasted_iota(jnp.int32, (BN, 1), 0) + j * BN
    out_ref[0] = jnp.where(rows == hb, q_ref[0], 0.0)


def _boundary(h_new, query):
    return pl.pallas_call(
        _boundary_body,
        grid=(B, N // BN),
        in_specs=[
            pl.BlockSpec(memory_space=pltpu.SMEM),
            pl.BlockSpec((1, 1, D), lambda b, j: (b, 0, 0)),
        ],
        out_specs=pl.BlockSpec((1, BN, D), lambda b, j: (b, j, 0)),
        out_shape=jax.ShapeDtypeStruct((B, N, D), jnp.float32),
    )(h_new, query.reshape(B, 1, D))


def _make_sc_layer(layer):
    mesh = plsc.VectorSubcoreMesh(core_axis_name="c", subcore_axis_name="s",
                                  num_cores=NC, num_subcores=NS)

    def body(x_hbm, comb_hbm, src_hbm, dst_hbm, et_hbm, etm_hbm, out_hbm,
             aggs, srcv, dstv, ctv, ttv, gidxv, xrows, crows, zbuf, sem):
        c = lax.axis_index("c")
        s = lax.axis_index("s")

        def zfill(i, carry):
            for j in range(D // 16):
                zbuf[i, pl.ds(j * 16, 16)] = jnp.zeros((16,), jnp.float32)
            return carry
        lax.fori_loop(0, ZROWS, zfill, 0)
        for k in range(ROWS_PT // ZROWS):
            pltpu.sync_copy(
                zbuf, aggs.at[pl.ds(s * ROWS_PT + k * ZROWS, ZROWS)])

        @pl.when(s == 0)
        def _():
            pltpu.sync_copy(zbuf.at[pl.ds(0, ROWS_REM)],
                            aggs.at[pl.ds(NS * ROWS_PT, ROWS_REM)])
        plsc.subcore_barrier()

        base = s * EPT

        def chunk_body(k, carry):
            off = base + k * CHUNK
            pltpu.sync_copy(src_hbm.at[pl.ds(off, CHUNK)], srcv)
            pltpu.sync_copy(dst_hbm.at[pl.ds(off, CHUNK)], dstv)
            pltpu.sync_copy(et_hbm.at[pl.ds(off, CHUNK)], ctv)
            pltpu.sync_copy(etm_hbm.at[pl.ds(off, CHUNK)], ttv)
            for j in range(CHUNK // 16):
                sl = pl.ds(j * 16, 16)
                ctv[sl] = ctv[sl] * HIST + ttv[sl] + (layer * R * HIST)
                gidxv[sl] = srcv[sl] + (2 * c) * N
            pltpu.async_copy(comb_hbm.at[ctv], crows, sem).wait()
            for bi in range(2):
                if bi == 1:
                    for j in range(CHUNK // 16):
                        sl = pl.ds(j * 16, 16)
                        gidxv[sl] = gidxv[sl] + N
                        dstv[sl] = dstv[sl] + N
                pltpu.async_copy(x_hbm.at[gidxv], xrows, sem).wait()

                def mrow(i, carry2):
                    for j2 in range(D // 16):
                        sl2 = pl.ds(j2 * 16, 16)
                        xrows[i, sl2] = xrows[i, sl2] * crows[i, sl2]
                    return carry2
                lax.fori_loop(0, CHUNK, mrow, 0)
                pltpu.sync_copy(xrows, aggs.at[dstv], add=True)
            return carry
        lax.fori_loop(0, NCHUNKS, chunk_body, 0)

        plsc.subcore_barrier()
        pltpu.sync_copy(
            aggs.at[pl.ds(s * ROWS_PT, ROWS_PT)],
            out_hbm.at[pl.ds(c * 2 * N + s * ROWS_PT, ROWS_PT)])

        @pl.when(s == 0)
        def _():
            pltpu.sync_copy(
                aggs.at[pl.ds(NS * ROWS_PT, ROWS_REM)],
                out_hbm.at[pl.ds(c * 2 * N + NS * ROWS_PT, ROWS_REM)])

    return pl.kernel(
        body,
        out_type=jax.ShapeDtypeStruct((B * N, D), jnp.float32),
        mesh=mesh,
        compiler_params=pltpu.CompilerParams(use_tc_tiling_on_sc=False),
        scratch_types=[
            pltpu.VMEM_SHARED((2 * N, D), jnp.float32),
            pltpu.VMEM((CHUNK,), jnp.int32),
            pltpu.VMEM((CHUNK,), jnp.int32),
            pltpu.VMEM((CHUNK,), jnp.int32),
            pltpu.VMEM((CHUNK,), jnp.int32),
            pltpu.VMEM((CHUNK,), jnp.int32),
            pltpu.VMEM((CHUNK, D), jnp.float32),
            pltpu.VMEM((CHUNK, D), jnp.float32),
            pltpu.VMEM((ZROWS, D), jnp.float32),
            pltpu.SemaphoreType.DMA,
        ],
    )


BM = 2000


def _dense2_body(x_ref, a_ref, wa_ref, wb_ref, b_ref, out_ref):
    acc = jnp.dot(x_ref[...], wa_ref[...], preferred_element_type=jnp.float32)
    acc += jnp.dot(a_ref[...], wb_ref[...], preferred_element_type=jnp.float32)
    out_ref[...] = jnp.maximum(acc + b_ref[0, :][None, :], 0.0)


def _dense3_body(x_ref, a_ref, x0_ref, wa_ref, wb_ref, b_ref, out_ref):
    acc = jnp.dot(x_ref[...], wa_ref[...], preferred_element_type=jnp.float32)
    acc += jnp.dot(a_ref[...], wb_ref[...], preferred_element_type=jnp.float32)
    acc += jnp.dot(x0_ref[...], wb_ref[...], preferred_element_type=jnp.float32)
    out_ref[...] = jnp.maximum(acc + b_ref[0, :][None, :], 0.0)


def _dense(x, agg, wa, wb, bias, x0=None):
    row_spec = pl.BlockSpec((BM, D), lambda i: (i, 0))
    full_spec = pl.BlockSpec((D, D), lambda i: (0, 0))
    bias_spec = pl.BlockSpec((1, D), lambda i: (0, 0))
    if x0 is None:
        return pl.pallas_call(
            _dense2_body,
            grid=(B * N // BM,),
            in_specs=[row_spec, row_spec, full_spec, full_spec, bias_spec],
            out_specs=row_spec,
            out_shape=jax.ShapeDtypeStruct((B * N, D), jnp.float32),
        )(x, agg, wa, wb, bias.reshape(1, D))
    return pl.pallas_call(
        _dense3_body,
        grid=(B * N // BM,),
        in_specs=[row_spec, row_spec, row_spec, full_spec, full_spec,
                  bias_spec],
        out_specs=row_spec,
        out_shape=jax.ShapeDtypeStruct((B * N, D), jnp.float32),
    )(x, agg, x0, wa, wb, bias.reshape(1, D))


def _final_body(t_ref, x2_ref, q_ref, m0a_ref, m0b_ref, mb0_ref,
                m1_ref, mb1_ref, out_ref, feat_ref):
    b = pl.program_id(0)

    def gather(i, carry):
        idx = t_ref[b, i]
        feat_ref[pl.ds(i, 1), :] = x2_ref[0, pl.ds(idx, 1), :]
        return carry
    lax.fori_loop(0, NEG, gather, 0)

    hdd = jnp.dot(feat_ref[...], m0a_ref[...],
                  preferred_element_type=jnp.float32)
    hdd += jnp.dot(q_ref[0], m0b_ref[...],
                   preferred_element_type=jnp.float32)
    hdd = jnp.maximum(hdd + mb0_ref[0, :][None, :], 0.0)
    score = jnp.dot(hdd, m1_ref[...], preferred_element_type=jnp.float32)
    out_ref[0] = score + mb1_ref[0, 0]


def _final(x2, t_new, query, m0a, m0b, mb0, m1, mb1):
    out = pl.pallas_call(
        _final_body,
        grid=(B,),
        in_specs=[
            pl.BlockSpec(memory_space=pltpu.SMEM),
            pl.BlockSpec((1, N, D), lambda b: (b, 0, 0)),
            pl.BlockSpec((1, 1, D), lambda b: (b, 0, 0)),
            pl.BlockSpec((D, 2 * D), lambda b: (0, 0)),
            pl.BlockSpec((D, 2 * D), lambda b: (0, 0)),
            pl.BlockSpec((1, 2 * D), lambda b: (0, 0)),
            pl.BlockSpec((2 * D, 1), lambda b: (0, 0)),
            pl.BlockSpec((1, 1), lambda b: (0, 0)),
        ],
        out_specs=pl.BlockSpec((1, NEGP, 1), lambda b: (b, 0, 0)),
        out_shape=jax.ShapeDtypeStruct((B, NEGP, 1), jnp.float32),
        scratch_shapes=[pltpu.VMEM((NEGP, D), jnp.float32)],
    )(t_new, x2, query.reshape(B, 1, D), m0a, m0b, mb0.reshape(1, 2 * D), m1,
      mb1.reshape(1, 1))
    return out[:, :NEG, 0]


def kernel(edge_index, edge_type, edge_time, query_triple, query_table,
           rel0, rel1, tw, tb, W0, b0, W1, b1, M0, mb0, M1, mb1):
    h_index = query_triple[..., 0]
    r_index = query_triple[..., 1]
    t_index = query_triple[..., 2]
    is_t_neg = jnp.all(h_index == h_index[:, :1], axis=-1, keepdims=True)
    h_new = jnp.where(is_t_neg, h_index, t_index)
    t_new = jnp.where(is_t_neg, t_index, h_index)
    r_new = jnp.where(is_t_neg, r_index, r_index + R // 2)

    src = edge_index[0]
    dst = edge_index[1]

    comb, query = _prep(rel0, rel1, tw, tb, query_table, r_new[:, 0])
    x0 = _boundary(h_new[:, 0], query)
    x0f = x0.reshape(B * N, D)

    w0a, w0b = W0[:D], W0[D:]
    w1a, w1b = W1[:D], W1[D:]

    sc0 = _make_sc_layer(0)
    sc1 = _make_sc_layer(1)

    agg1 = sc0(x0f, comb, src, dst, edge_type, edge_time)
    x1 = _dense(x0f, agg1, w0a + w0b, w0b, b0)
    agg2 = sc1(x1, comb, src, dst, edge_type, edge_time)
    x2 = _dense(x1, agg2, w1a, w1b, b1, x0=x0f)

    score = _final(x2.reshape(B, N, D), t_new, query, M0[:D], M0[D:],
                   mb0, M1, mb1)
    return score


# speedup vs baseline: 74.5941x; 3.2153x over previous
import functools

import jax
import jax.numpy as jnp
from jax import lax
from jax.experimental import pallas as pl
from jax.experimental.pallas import tpu as pltpu
from jax.experimental.pallas import tpu_sc as plsc

N = 10000
E = 160000
D = 64
R = 100
B = 4
NEG = 33
HIST = 10

NC = 2
NS = 16
CH = 96
NCHG = 1696
EPAD = NCHG * CH
CPT = NCHG // NS
TRASH = 2 * N
AROWS = 2 * N + 8
ROWS_PT = 1248
ROWS_REM = 2 * N - NS * ROWS_PT
ZROWS = 48
NCOMB = 2 * R * HIST
NCL = R * HIST

NEGP = 40
BM = 2000


def _prep_body(rel0_ref, rel1_ref, tw_ref, tb_ref, qt_ref, rn_ref,
               w0_ref, w1_ref,
               comb_ref, q_ref, xq_ref, qw0_ref, qw1_ref):
    t_ids = lax.broadcasted_iota(jnp.int32, (HIST, D), 0).astype(jnp.float32)
    te = jnp.cos(t_ids * tw_ref[0, :][None, :] + tb_ref[0, :][None, :])
    comb_ref[0] = rel0_ref[...][:, None, :] + te[None, :, :]
    comb_ref[1] = rel1_ref[...][:, None, :] + te[None, :, :]
    rids = lax.broadcasted_iota(jnp.int32, (B, R), 1)
    onehot = (rids == rn_ref[...]).astype(jnp.float32)
    q = jnp.dot(onehot, qt_ref[...], preferred_element_type=jnp.float32)
    q_ref[...] = q
    xq_ref[...] = jnp.concatenate([jnp.zeros((B, D), jnp.float32), q], axis=0)
    w0s = w0_ref[0:D, :] + w0_ref[D:2 * D, :]
    qw0_ref[...] = jnp.dot(q, w0s, preferred_element_type=jnp.float32)
    qw1_ref[...] = jnp.dot(q, w1_ref[D:2 * D, :],
                           preferred_element_type=jnp.float32)


def _prep(rel0, rel1, tw, tb, query_table, rn0, W0, W1):
    comb4, query, xq, qw0, qw1 = pl.pallas_call(
        _prep_body,
        out_shape=(
            jax.ShapeDtypeStruct((2, R, HIST, D), jnp.float32),
            jax.ShapeDtypeStruct((B, D), jnp.float32),
            jax.ShapeDtypeStruct((2 * B, D), jnp.float32),
            jax.ShapeDtypeStruct((B, D), jnp.float32),
            jax.ShapeDtypeStruct((B, D), jnp.float32),
        ),
    )(rel0, rel1, tw.reshape(1, D), tb.reshape(1, D), query_table,
      rn0.reshape(B, 1), W0, W1)
    return comb4.reshape(NCOMB, D), query, xq, qw0, qw1


def _sc_mesh():
    return plsc.VectorSubcoreMesh(core_axis_name="c", subcore_axis_name="s",
                                  num_cores=NC, num_subcores=NS)


def _sc_params():
    return pltpu.CompilerParams(use_tc_tiling_on_sc=False,
                                needs_layout_passes=False)


def _zero_aggs(s, zbuf, aggs):
    def zfill(i, carry):
        for j in range(D // 16):
            zbuf[i, pl.ds(j * 16, 16)] = jnp.zeros((16,), jnp.float32)
        return carry
    lax.fori_loop(0, ZROWS, zfill, 0)
    for k in range(ROWS_PT // ZROWS):
        pltpu.sync_copy(zbuf, aggs.at[pl.ds(s * ROWS_PT + k * ZROWS, ZROWS)])

    @pl.when(s == 0)
    def _():
        pltpu.sync_copy(zbuf.at[pl.ds(0, ROWS_REM)],
                        aggs.at[pl.ds(NS * ROWS_PT, ROWS_REM)])
    plsc.subcore_barrier()


def _copy_out(c, s, aggs, out_hbm):
    plsc.subcore_barrier()
    pltpu.sync_copy(
        aggs.at[pl.ds(s * ROWS_PT, ROWS_PT)],
        out_hbm.at[pl.ds(c * 2 * N + s * ROWS_PT, ROWS_PT)])

    @pl.when(s == 0)
    def _():
        pltpu.sync_copy(
            aggs.at[pl.ds(NS * ROWS_PT, ROWS_REM)],
            out_hbm.at[pl.ds(c * 2 * N + NS * ROWS_PT, ROWS_REM)])


def _mul_rows(xr, crows):
    @plsc.parallel_loop(0, CH, step=1, unroll=8)
    def _(i):
        for j2 in range(D // 16):
            sl2 = pl.ds(j2 * 16, 16)
            xr[i, sl2] = xr[i, sl2] * crows[i, sl2]


def _make_sc_layer(layer):
    def body(x_hbm, comb_hbm, edata_hbm, aux_hbm, out_hbm,
             aggs, zbuf, hvv,
             ebufA, ctvA, gidx0A, gidx1A, lidx0A, lidx1A, fb0A, fb1A,
             xrA, crowsA,
             ebufB, ctvB, gidx0B, gidx1B, lidx0B, lidx1B, fb0B, fb1B,
             xrB, crowsB,
             semA, semB):
        c = lax.axis_index("c")
        s = lax.axis_index("s")
        bufA = (ebufA, ctvA, gidx0A, gidx1A, lidx0A, lidx1A, fb0A, fb1A,
                xrA, crowsA, semA)
        bufB = (ebufB, ctvB, gidx0B, gidx1B, lidx0B, lidx1B, fb0B, fb1B,
                xrB, crowsB, semB)
        _zero_aggs(s, zbuf, aggs)
        if layer == 0:
            pltpu.sync_copy(aux_hbm, hvv)
            hv0 = plsc.load_gather(hvv, [jnp.full((16,), 2 * c, jnp.int32)])
            hv1 = plsc.load_gather(hvv,
                                   [jnp.full((16,), 2 * c + 1, jnp.int32)])
        base = s * CPT

        def prep_chunk(buf):
            (ebuf, ctv, gidx0, gidx1, lidx0, lidx1, fb0, fb1, xr, crows,
             sem) = buf
            for j in range(CH // 16):
                sl = pl.ds(j * 16, 16)
                ctv[sl] = (ebuf[2, sl] * HIST + ebuf[3, sl] + layer * NCL)
                dv = ebuf[1, sl]
                lidx0[sl] = jnp.minimum(dv, TRASH)
                lidx1[sl] = jnp.minimum(dv + N, TRASH)
            if layer == 0:
                cnt0 = jnp.zeros((16,), jnp.int32)
                cnt1 = jnp.zeros((16,), jnp.int32)
                for j in range(CH // 16):
                    sl = pl.ds(j * 16, 16)
                    sv = ebuf[0, sl]
                    m0 = sv == hv0
                    m1 = sv == hv1
                    gidx0[sl] = jnp.where(m0, 2 * c + B, 0)
                    gidx1[sl] = jnp.where(m1, 2 * c + 1 + B, 0)
                    cnt0 = cnt0 + jnp.where(m0, 1, 0)
                    cnt1 = cnt1 + jnp.where(m1, 1, 0)
                return (jnp.sum(cnt0), jnp.sum(cnt1))
            for j in range(CH // 16):
                sl = pl.ds(j * 16, 16)
                sv = ebuf[0, sl]
                gidx0[sl] = sv + (2 * c) * N
                gidx1[sl] = sv + (2 * c + 1) * N
            return (pltpu.async_copy(aux_hbm.at[gidx0], fb0, sem),
                    pltpu.async_copy(aux_hbm.at[gidx1], fb1, sem))

        def process(buf, marks):
            (ebuf, ctv, gidx0, gidx1, lidx0, lidx1, fb0, fb1, xr, crows,
             sem) = buf
            if layer == 0:
                nm0, nm1 = marks
            else:
                marks[0].wait()
                marks[1].wait()
                cnt0 = jnp.zeros((16,), jnp.int32)
                cnt1 = jnp.zeros((16,), jnp.int32)
                for j in range(CH // 16):
                    sl = pl.ds(j * 16, 16)
                    cnt0 = cnt0 + fb0[sl]
                    cnt1 = cnt1 + fb1[sl]
                nm0 = jnp.sum(cnt0)
                nm1 = jnp.sum(cnt1)

            @pl.when(nm0 + nm1 > 0)
            def _():
                pltpu.async_copy(comb_hbm.at[ctv], crows, sem).wait()

                @pl.when(nm0 > 0)
                def _():
                    pltpu.async_copy(x_hbm.at[gidx0], xr, sem).wait()
                    _mul_rows(xr, crows)
                    pltpu.sync_copy(xr, aggs.at[lidx0], add=True)

                @pl.when(nm1 > 0)
                def _():
                    pltpu.async_copy(x_hbm.at[gidx1], xr, sem).wait()
                    _mul_rows(xr, crows)
                    pltpu.sync_copy(xr, aggs.at[lidx1], add=True)

        NIT = CPT // 2

        def pipe_body(k2, carry):
            g0 = base + 2 * k2
            dA = pltpu.async_copy(edata_hbm.at[g0], ebufA, semA)
            dB = pltpu.async_copy(edata_hbm.at[g0 + 1], ebufB, semB)
            dA.wait()
            mA = prep_chunk(bufA)
            dB.wait()
            mB = prep_chunk(bufB)
            process(bufA, mA)
            process(bufB, mB)
            return carry
        lax.fori_loop(0, NIT, pipe_body, 0)
        _copy_out(c, s, aggs, out_hbm)

    ibuf = [
        pltpu.VMEM((4, CH), jnp.int32),
        pltpu.VMEM((CH,), jnp.int32),
        pltpu.VMEM((CH,), jnp.int32),
        pltpu.VMEM((CH,), jnp.int32),
        pltpu.VMEM((CH,), jnp.int32),
        pltpu.VMEM((CH,), jnp.int32),
        pltpu.VMEM((CH,), jnp.int32),
        pltpu.VMEM((CH,), jnp.int32),
        pltpu.VMEM((CH, D), jnp.float32),
        pltpu.VMEM((CH, D), jnp.float32),
    ]
    return pl.kernel(
        body,
        out_type=jax.ShapeDtypeStruct((B * N, D), jnp.float32),
        mesh=_sc_mesh(),
        compiler_params=_sc_params(),
        scratch_types=[
            pltpu.VMEM_SHARED((AROWS, D), jnp.float32),
            pltpu.VMEM((ZROWS, D), jnp.float32),
            pltpu.VMEM((16,), jnp.int32),
        ] + ibuf + ibuf + [
            pltpu.SemaphoreType.DMA,
            pltpu.SemaphoreType.DMA,
        ],
    )


def _dense1_body(h_ref, qw_ref, a_ref, wb_ref, b_ref, out_ref, nz_ref):
    b = pl.program_id(0)
    j = pl.program_id(1)
    rows = lax.broadcasted_iota(jnp.int32, (BM, 1), 0) + j * BM
    acc = jnp.dot(a_ref[...], wb_ref[...], preferred_element_type=jnp.float32)
    acc += jnp.where(rows == h_ref[b], qw_ref[0], 0.0)
    x1 = jnp.maximum(acc + b_ref[0, :][None, :], 0.0)
    out_ref[...] = x1
    nz_ref[...] = (jnp.max(x1, axis=1, keepdims=True) > 0).astype(jnp.int32)


def _row_spec():
    return pl.BlockSpec((BM, D), lambda b, j: (b * (N // BM) + j, 0))


def _dense1(h_new, qw0, agg, wb, bias):
    return pl.pallas_call(
        _dense1_body,
        grid=(B, N // BM),
        in_specs=[
            pl.BlockSpec(memory_space=pltpu.SMEM),
            pl.BlockSpec((1, 1, D), lambda b, j: (b, 0, 0)),
            _row_spec(),
            pl.BlockSpec((D, D), lambda b, j: (0, 0)),
            pl.BlockSpec((1, D), lambda b, j: (0, 0)),
        ],
        out_specs=(
            _row_spec(),
            pl.BlockSpec((BM, 1), lambda b, j: (b * (N // BM) + j, 0)),
        ),
        out_shape=(
            jax.ShapeDtypeStruct((B * N, D), jnp.float32),
            jax.ShapeDtypeStruct((B * N, 1), jnp.int32),
        ),
    )(h_new, qw0.reshape(B, 1, D), agg, wb, bias.reshape(1, D))


def _final_body(t_ref, h_ref, x1_ref, a_ref, qw_ref, w1a_ref, w1b_ref,
                b1_ref, q_ref, m0a_ref, m0b_ref, mb0_ref, m1_ref, mb1_ref,
                out_ref, f1_ref, f2_ref, mk_ref):
    b = pl.program_id(0)
    hb = h_ref[b]

    def gather(i, carry):
        idx = t_ref[b, i]
        f1_ref[pl.ds(i, 1), :] = x1_ref[0, pl.ds(idx, 1), :]
        f2_ref[pl.ds(i, 1), :] = a_ref[0, pl.ds(idx, 1), :]
        mk_ref[pl.ds(i, 1), :] = jnp.where(idx == hb, 1.0, 0.0)[None, None]
        return carry
    lax.fori_loop(0, NEG, gather, 0)

    x2r = jnp.dot(f1_ref[...], w1a_ref[...],
                  preferred_element_type=jnp.float32)
    x2r += jnp.dot(f2_ref[...], w1b_ref[...],
                   preferred_element_type=jnp.float32)
    x2r += mk_ref[...] * qw_ref[0]
    x2r = jnp.maximum(x2r + b1_ref[0, :][None, :], 0.0)
    hdd = jnp.dot(x2r, m0a_ref[...], preferred_element_type=jnp.float32)
    hdd += jnp.dot(q_ref[0], m0b_ref[...],
                   preferred_element_type=jnp.float32)
    hdd = jnp.maximum(hdd + mb0_ref[0, :][None, :], 0.0)
    score = jnp.dot(hdd, m1_ref[...], preferred_element_type=jnp.float32)
    out_ref[0] = score + mb1_ref[0, 0]


def _final(x1, agg2, t_new, h_new, qw1, w1a, w1b, b1, query,
           m0a, m0b, mb0, m1, mb1):
    def full2(s0, s1):
        return pl.BlockSpec((s0, s1), lambda b: (0, 0))
    out = pl.pallas_call(
        _final_body,
        grid=(B,),
        in_specs=[
            pl.BlockSpec(memory_space=pltpu.SMEM),
            pl.BlockSpec(memory_space=pltpu.SMEM),
            pl.BlockSpec((1, N, D), lambda b: (b, 0, 0)),
            pl.BlockSpec((1, N, D), lambda b: (b, 0, 0)),
            pl.BlockSpec((1, 1, D), lambda b: (b, 0, 0)),
            full2(D, D),
            full2(D, D),
            full2(1, D),
            pl.BlockSpec((1, 1, D), lambda b: (b, 0, 0)),
            full2(D, 2 * D),
            full2(D, 2 * D),
            full2(1, 2 * D),
            full2(2 * D, 1),
            full2(1, 1),
        ],
        out_specs=pl.BlockSpec((1, NEGP, 1), lambda b: (b, 0, 0)),
        out_shape=jax.ShapeDtypeStruct((B, NEGP, 1), jnp.float32),
        scratch_shapes=[
            pltpu.VMEM((NEGP, D), jnp.float32),
            pltpu.VMEM((NEGP, D), jnp.float32),
            pltpu.VMEM((NEGP, 1), jnp.float32),
        ],
    )(t_new, h_new, x1, agg2, qw1.reshape(B, 1, D), w1a, w1b,
      b1.reshape(1, D), query.reshape(B, 1, D), m0a, m0b,
      mb0.reshape(1, 2 * D), m1, mb1.reshape(1, 1))
    return out[:, :NEG, 0]


def kernel(edge_index, edge_type, edge_time, query_triple, query_table,
           rel0, rel1, tw, tb, W0, b0, W1, b1, M0, mb0, M1, mb1):
    h_index = query_triple[..., 0]
    r_index = query_triple[..., 1]
    t_index = query_triple[..., 2]
    is_t_neg = jnp.all(h_index == h_index[:, :1], axis=-1, keepdims=True)
    h_new = jnp.where(is_t_neg, h_index, t_index)
    t_new = jnp.where(is_t_neg, t_index, h_index)
    r_new = jnp.where(is_t_neg, r_index, r_index + R // 2)

    pad = EPAD - E
    srcp = jnp.pad(edge_index[0], (0, pad))
    dstp = jnp.pad(edge_index[1], (0, pad), constant_values=TRASH)
    etp = jnp.pad(edge_type, (0, pad))
    etmp = jnp.pad(edge_time, (0, pad))
    edata = (jnp.stack([srcp, dstp, etp, etmp], axis=0)
             .reshape(4, NCHG, CH).transpose(1, 0, 2))
    hv = jnp.pad(h_new[:, 0], (0, 16 - B))

    comb, query, xq, qw0, qw1 = _prep(rel0, rel1, tw, tb, query_table,
                                      r_new[:, 0], W0, W1)

    sc0 = _make_sc_layer(0)
    sc1 = _make_sc_layer(1)

    agg1 = sc0(xq, comb, edata, hv)
    x1, nzf = _dense1(h_new[:, 0], qw0, agg1, W0[D:], b0)
    agg2 = sc1(x1, comb, edata, nzf.reshape(B * N))
    score = _final(x1.reshape(B, N, D), agg2.reshape(B, N, D), t_new,
                   h_new[:, 0], qw1, W1[:D], W1[D:], b1, query,
                   M0[:D], M0[D:], mb0, M1, mb1)
    return score
